# Initial kernel scaffold; baseline (speedup 1.0000x reference)
import functools

import jax
import jax.numpy as jnp
from jax import lax
from jax.experimental import pallas as pl
from jax.experimental.pallas import tpu as pltpu
from jax.experimental.pallas import tpu_sc as plsc

VOCAB = 1000000
DIM = 64
BATCH = 16384
NSAMP = 20

NUM_CORES = 2
NUM_SUBCORES = 16
NW = NUM_CORES * NUM_SUBCORES
B_PER_W = BATCH // NW
CHUNK = 64
NCHUNK = B_PER_W // CHUNK
SROWS = CHUNK * NSAMP
NSTREAM = SROWS // 128


def _compute_chunk(item_rows, samp_rows, out_v):
    iota = lax.iota(jnp.int32, 16)
    for g in range(CHUNK // 16):
        row0 = g * 16
        item_row_idx = iota + row0
        samp_row_base = (iota + row0) * NSAMP

        def dbody(d, acc):
            dcol = jnp.full((16,), d, dtype=jnp.int32)
            item_col = plsc.load_gather(item_rows, [item_row_idx, dcol])
            new_acc = []
            for l in range(NSAMP):
                scol = plsc.load_gather(samp_rows, [samp_row_base + l, dcol])
                new_acc.append(acc[l] + item_col * scol)
            return tuple(new_acc)

        acc = lax.fori_loop(
            0, DIM, dbody,
            tuple(jnp.zeros((16,), jnp.float32) for _ in range(NSAMP)))
        for l in range(NSAMP):
            plsc.store_scatter(out_v, [samp_row_base + l], acc[l])


def _item2vec_sc(items, samples_flat, embeddings):
    mesh = plsc.VectorSubcoreMesh(
        core_axis_name="c", subcore_axis_name="s",
        num_cores=NUM_CORES, num_subcores=NUM_SUBCORES)

    @functools.partial(
        pl.kernel,
        out_type=jax.ShapeDtypeStruct((BATCH * NSAMP,), jnp.float32),
        mesh=mesh,
        scratch_types=[
            pltpu.VMEM((CHUNK,), jnp.int32),
            pltpu.VMEM((NSTREAM, 128), jnp.int32),
            pltpu.VMEM((CHUNK, DIM), jnp.float32),
            pltpu.VMEM((SROWS, DIM), jnp.float32),
            pltpu.VMEM((SROWS,), jnp.float32),
            pltpu.SemaphoreType.DMA,
        ],
    )
    def k(items_hbm, samples_hbm, emb_hbm, out_hbm,
          iidx_v, sidx_v, irows_v, srows_v, out_v, sem):
        wid = lax.axis_index("s") * NUM_CORES + lax.axis_index("c")
        wbase = wid * B_PER_W

        def chunk_body(c, carry):
            base = wbase + c * CHUNK
            pltpu.sync_copy(items_hbm.at[pl.ds(base, CHUNK)], iidx_v)
            pltpu.sync_copy(
                samples_hbm.at[pl.ds(base * NSAMP, SROWS)],
                sidx_v.reshape(SROWS))
            copies = [pltpu.make_async_copy(emb_hbm.at[iidx_v], irows_v, sem)]
            for j in range(NSTREAM):
                copies.append(pltpu.make_async_copy(
                    emb_hbm.at[sidx_v.at[j]],
                    srows_v.at[pl.ds(j * 128, 128)], sem))
            for cp in copies:
                cp.start()
            for cp in copies:
                cp.wait()
            _compute_chunk(irows_v, srows_v, out_v)
            pltpu.sync_copy(out_v, out_hbm.at[pl.ds(base * NSAMP, SROWS)])
            return carry

        lax.fori_loop(0, NCHUNK, chunk_body, 0)

    return k(items, samples_flat, embeddings)


@jax.jit
def kernel(items, samples, embeddings):
    scores_flat = _item2vec_sc(items, samples.reshape(-1), embeddings)
    return scores_flat.reshape(BATCH, NSAMP)


# speedup vs baseline: 5.6981x; 5.6981x over previous
import functools

import jax
import jax.numpy as jnp
from jax import lax
from jax.experimental import pallas as pl
from jax.experimental.pallas import tpu as pltpu
from jax.experimental.pallas import tpu_sc as plsc

VOCAB = 1000000
DIM = 64
BATCH = 16384
NSAMP = 20

NUM_CORES = 2
NUM_SUBCORES = 16
NW = NUM_CORES * NUM_SUBCORES
B_PER_W = BATCH // NW
CHUNK = 64
NCHUNK = B_PER_W // CHUNK
SROWS = CHUNK * NSAMP
NSTREAM = SROWS // 128


def _compute_chunk(item_rows, samp_rows, out_v):
    iota = lax.iota(jnp.int32, 16)
    for g in range(CHUNK // 16):
        row0 = g * 16
        item_row_idx = iota + row0
        samp_row_base = (iota + row0) * NSAMP

        def dbody(d, acc):
            dcol = jnp.full((16,), d, dtype=jnp.int32)
            item_col = plsc.load_gather(item_rows, [item_row_idx, dcol])
            new_acc = []
            for l in range(NSAMP):
                scol = plsc.load_gather(samp_rows, [samp_row_base + l, dcol])
                new_acc.append(acc[l] + item_col * scol)
            return tuple(new_acc)

        acc = lax.fori_loop(
            0, DIM, dbody,
            tuple(jnp.zeros((16,), jnp.float32) for _ in range(NSAMP)))
        for l in range(NSAMP):
            plsc.store_scatter(out_v, [samp_row_base + l], acc[l])


def _item2vec_sc(items, samples_flat, embeddings):
    mesh = plsc.VectorSubcoreMesh(
        core_axis_name="c", subcore_axis_name="s",
        num_cores=NUM_CORES, num_subcores=NUM_SUBCORES)

    @functools.partial(
        pl.kernel,
        out_type=jax.ShapeDtypeStruct((BATCH * NSAMP,), jnp.float32),
        mesh=mesh,
        scratch_types=[
            pltpu.VMEM((CHUNK,), jnp.int32),
            pltpu.VMEM((SROWS,), jnp.int32),
            pltpu.VMEM((CHUNK, DIM), jnp.float32),
            pltpu.VMEM((SROWS, DIM), jnp.float32),
            pltpu.VMEM((SROWS,), jnp.float32),
            pltpu.SemaphoreType.DMA,
        ],
        compiler_params=pltpu.CompilerParams(
            needs_layout_passes=False, use_tc_tiling_on_sc=False),
    )
    def k(items_hbm, samples_hbm, emb_hbm, out_hbm,
          iidx_v, sidx_v, irows_v, srows_v, out_v, sem):
        wid = lax.axis_index("s") * NUM_CORES + lax.axis_index("c")
        wbase = wid * B_PER_W

        def chunk_body(c, carry):
            base = wbase + c * CHUNK
            pltpu.sync_copy(items_hbm.at[pl.ds(base, CHUNK)], iidx_v)
            pltpu.sync_copy(
                samples_hbm.at[pl.ds(base * NSAMP, SROWS)], sidx_v)
            copies = [pltpu.make_async_copy(emb_hbm.at[iidx_v], irows_v, sem)]
            for j in range(NSTREAM):
                copies.append(pltpu.make_async_copy(
                    emb_hbm.at[sidx_v.at[pl.ds(j * 128, 128)]],
                    srows_v.at[pl.ds(j * 128, 128)], sem))
            for cp in copies:
                cp.start()
            for cp in copies:
                cp.wait()
            _compute_chunk(irows_v, srows_v, out_v)
            pltpu.sync_copy(out_v, out_hbm.at[pl.ds(base * NSAMP, SROWS)])
            return carry

        lax.fori_loop(0, NCHUNK, chunk_body, 0)

    return k(items, samples_flat, embeddings)


@jax.jit
def kernel(items, samples, embeddings):
    scores_flat = _item2vec_sc(items, samples.reshape(-1), embeddings)
    return scores_flat.reshape(BATCH, NSAMP)


# speedup vs baseline: 5.9831x; 1.0500x over previous
import functools

import jax
import jax.numpy as jnp
from jax import lax
from jax.experimental import pallas as pl
from jax.experimental.pallas import tpu as pltpu
from jax.experimental.pallas import tpu_sc as plsc

VOCAB = 1000000
DIM = 64
BATCH = 16384
NSAMP = 20

NUM_CORES = 2
NUM_SUBCORES = 16
NW = NUM_CORES * NUM_SUBCORES
B_PER_W = BATCH // NW
CHUNK = 32
NCHUNK = B_PER_W // CHUNK
SROWS = CHUNK * NSAMP
NSTREAM = SROWS // 128


def _compute_chunk(item_rows, samp_rows, out_v):
    iota = lax.iota(jnp.int32, 16)
    for g in range(CHUNK // 16):
        row0 = g * 16
        item_row_idx = iota + row0
        samp_rows_idx = [(iota + row0) * NSAMP + l for l in range(NSAMP)]

        def dbody(i, acc, item_row_idx=item_row_idx,
                  samp_rows_idx=samp_rows_idx):
            new_acc = list(acc)
            for u in range(2):
                d = i * 2 + u
                dcol = jnp.full((16,), d, dtype=jnp.int32)
                item_col = plsc.load_gather(item_rows, [item_row_idx, dcol])
                for l in range(NSAMP):
                    scol = plsc.load_gather(
                        samp_rows, [samp_rows_idx[l], dcol])
                    new_acc[l] = new_acc[l] + item_col * scol
            return tuple(new_acc)

        acc = lax.fori_loop(
            0, DIM // 2, dbody,
            tuple(jnp.zeros((16,), jnp.float32) for _ in range(NSAMP)))
        for l in range(NSAMP):
            plsc.store_scatter(out_v, [samp_rows_idx[l]], acc[l])


def _item2vec_sc(items, samples_flat, embeddings):
    mesh = plsc.VectorSubcoreMesh(
        core_axis_name="c", subcore_axis_name="s",
        num_cores=NUM_CORES, num_subcores=NUM_SUBCORES)

    @functools.partial(
        pl.kernel,
        out_type=jax.ShapeDtypeStruct((BATCH * NSAMP,), jnp.float32),
        mesh=mesh,
        scratch_types=[
            pltpu.VMEM((B_PER_W,), jnp.int32),
            pltpu.VMEM((B_PER_W * NSAMP,), jnp.int32),
            [pltpu.VMEM((CHUNK, DIM), jnp.float32)] * 2,
            [pltpu.VMEM((SROWS, DIM), jnp.float32)] * 2,
            [pltpu.VMEM((SROWS,), jnp.float32)] * 2,
            [pltpu.SemaphoreType.DMA] * 2,
            [pltpu.SemaphoreType.DMA] * 2,
        ],
        compiler_params=pltpu.CompilerParams(
            needs_layout_passes=False, use_tc_tiling_on_sc=False),
    )
    def k(items_hbm, samples_hbm, emb_hbm, out_hbm,
          iidx_v, sidx_v, irows, srows, outs, gsems, osems):
        wid = lax.axis_index("s") * NUM_CORES + lax.axis_index("c")
        wbase = wid * B_PER_W
        pltpu.sync_copy(items_hbm.at[pl.ds(wbase, B_PER_W)], iidx_v)
        pltpu.sync_copy(
            samples_hbm.at[pl.ds(wbase * NSAMP, B_PER_W * NSAMP)], sidx_v)

        def fire(c, b):
            base = pl.multiple_of(c * CHUNK, CHUNK)
            sbase = pl.multiple_of(c * SROWS, SROWS)
            pltpu.make_async_copy(
                emb_hbm.at[iidx_v.at[pl.ds(base, CHUNK)]],
                irows[b], gsems[b]).start()
            for j in range(NSTREAM):
                pltpu.make_async_copy(
                    emb_hbm.at[sidx_v.at[pl.ds(sbase + j * 128, 128)]],
                    srows[b].at[pl.ds(j * 128, 128)], gsems[b]).start()

        def drain(b):
            pltpu.make_async_copy(
                emb_hbm.at[iidx_v.at[pl.ds(0, CHUNK)]],
                irows[b], gsems[b]).wait()
            for j in range(NSTREAM):
                pltpu.make_async_copy(
                    emb_hbm.at[sidx_v.at[pl.ds(0, 128)]],
                    srows[b].at[pl.ds(j * 128, 128)], gsems[b]).wait()

        def out_wait(b):
            pltpu.make_async_copy(
                outs[b], out_hbm.at[pl.ds(0, SROWS)], osems[b]).wait()

        fire(0, 0)
        fire(1, 1)

        def pair_body(kk, carry):
            c0 = kk * 2

            @pl.when(kk > 0)
            def _():
                out_wait(0)
            _compute_after_drain0(c0)

            @pl.when(kk > 0)
            def _():
                out_wait(1)
            _compute_after_drain1(c0)
            return carry

        def _compute_after_drain0(c0):
            drain(0)
            _compute_chunk(irows[0], srows[0], outs[0])

            @pl.when(c0 + 2 < NCHUNK)
            def _():
                fire(c0 + 2, 0)
            obase = pl.multiple_of(wbase * NSAMP + c0 * SROWS, SROWS)
            pltpu.make_async_copy(
                outs[0], out_hbm.at[pl.ds(obase, SROWS)], osems[0]).start()

        def _compute_after_drain1(c0):
            drain(1)
            _compute_chunk(irows[1], srows[1], outs[1])

            @pl.when(c0 + 3 < NCHUNK)
            def _():
                fire(c0 + 3, 1)
            obase = pl.multiple_of(
                wbase * NSAMP + (c0 + 1) * SROWS, SROWS)
            pltpu.make_async_copy(
                outs[1], out_hbm.at[pl.ds(obase, SROWS)], osems[1]).start()

        lax.fori_loop(0, NCHUNK // 2, pair_body, 0)
        out_wait(0)
        out_wait(1)

    return k(items, samples_flat, embeddings)


@jax.jit
def kernel(items, samples, embeddings):
    scores_flat = _item2vec_sc(items, samples.reshape(-1), embeddings)
    return scores_flat.reshape(BATCH, NSAMP)


# speedup vs baseline: 6.1916x; 1.0348x over previous
import functools

import jax
import jax.numpy as jnp
from jax import lax
from jax.experimental import pallas as pl
from jax.experimental.pallas import tpu as pltpu
from jax.experimental.pallas import tpu_sc as plsc

VOCAB = 1000000
DIM = 64
BATCH = 16384
NSAMP = 20

NUM_CORES = 2
NUM_SUBCORES = 16
NW = NUM_CORES * NUM_SUBCORES
B_PER_W = BATCH // NW
CHUNK = 16
NCHUNK = B_PER_W // CHUNK
SROWS = CHUNK * NSAMP
SSTREAMS = (128, 128, 64)
NS_ALL = B_PER_W * NSAMP
LHALF = NSAMP // 2


def _compute_chunk(c, iidx_v, sidx_v, item_rows, samp_rows, out_v):
    iota = lax.iota(jnp.int32, 16)
    ibase = pl.multiple_of(c * CHUNK, CHUNK)
    sbase = pl.multiple_of(c * SROWS, SROWS)
    item_v = iidx_v[pl.ds(ibase, 16)]
    ic64 = (item_v & 1) << 6

    for h in range(2):
        srow = [iota * NSAMP + (h * LHALF + l) for l in range(LHALF)]
        sc64 = [
            (plsc.load_gather(sidx_v, [sbase + srow[l]]) & 1) << 6
            for l in range(LHALF)
        ]

        def dbody(i, acc, srow=srow, sc64=sc64):
            new_acc = list(acc)
            for u in range(2):
                d = i * 2 + u
                dcol = jnp.full((16,), d, dtype=jnp.int32)
                item_col = plsc.load_gather(item_rows, [iota, ic64 + dcol])
                for l in range(LHALF):
                    scol = plsc.load_gather(
                        samp_rows, [srow[l], sc64[l] + dcol])
                    new_acc[l] = new_acc[l] + item_col * scol
            return tuple(new_acc)

        acc = lax.fori_loop(
            0, DIM // 2, dbody,
            tuple(jnp.zeros((16,), jnp.float32) for _ in range(LHALF)))
        for l in range(LHALF):
            plsc.store_scatter(out_v, [srow[l]], acc[l])


def _item2vec_sc(items, samples_flat, emb2):
    mesh = plsc.VectorSubcoreMesh(
        core_axis_name="c", subcore_axis_name="s",
        num_cores=NUM_CORES, num_subcores=NUM_SUBCORES)

    @functools.partial(
        pl.kernel,
        out_type=jax.ShapeDtypeStruct((BATCH * NSAMP,), jnp.float32),
        mesh=mesh,
        scratch_types=[
            pltpu.VMEM((B_PER_W,), jnp.int32),
            pltpu.VMEM((NS_ALL,), jnp.int32),
            pltpu.VMEM((B_PER_W,), jnp.int32),
            pltpu.VMEM((NS_ALL,), jnp.int32),
            [pltpu.VMEM((CHUNK, 128), jnp.float32)] * 2,
            [pltpu.VMEM((SROWS, 128), jnp.float32)] * 2,
            [pltpu.VMEM((SROWS,), jnp.float32)] * 2,
            [pltpu.SemaphoreType.DMA] * 2,
            [pltpu.SemaphoreType.DMA] * 2,
        ],
        compiler_params=pltpu.CompilerParams(
            needs_layout_passes=False, use_tc_tiling_on_sc=True),
    )
    def k(items_hbm, samples_hbm, emb_hbm, out_hbm,
          iidx_v, sidx_v, ipix_v, spix_v, irows, srows, outs, gsems, osems):
        wid = lax.axis_index("s") * NUM_CORES + lax.axis_index("c")
        wbase = wid * B_PER_W
        pltpu.sync_copy(items_hbm.at[pl.ds(wbase, B_PER_W)], iidx_v)
        pltpu.sync_copy(
            samples_hbm.at[pl.ds(wbase * NSAMP, NS_ALL)], sidx_v)

        def halve(i, _, src, dst):
            off = pl.multiple_of(i * 16, 16)
            dst[pl.ds(off, 16)] = src[pl.ds(off, 16)] >> 1
            return _

        lax.fori_loop(0, B_PER_W // 16,
                      functools.partial(halve, src=iidx_v, dst=ipix_v), 0)
        lax.fori_loop(0, NS_ALL // 16,
                      functools.partial(halve, src=sidx_v, dst=spix_v), 0)

        def fire(c, b):
            ibase = pl.multiple_of(c * CHUNK, CHUNK)
            sbase = pl.multiple_of(c * SROWS, SROWS)
            pltpu.make_async_copy(
                emb_hbm.at[ipix_v.at[pl.ds(ibase, CHUNK)]],
                irows[b], gsems[b]).start()
            off = 0
            for n in SSTREAMS:
                pltpu.make_async_copy(
                    emb_hbm.at[spix_v.at[pl.ds(sbase + off, n)]],
                    srows[b].at[pl.ds(off, n)], gsems[b]).start()
                off += n

        def drain(b):
            pltpu.make_async_copy(
                emb_hbm.at[ipix_v.at[pl.ds(0, CHUNK)]],
                irows[b], gsems[b]).wait()
            off = 0
            for n in SSTREAMS:
                pltpu.make_async_copy(
                    emb_hbm.at[spix_v.at[pl.ds(0, n)]],
                    srows[b].at[pl.ds(off, n)], gsems[b]).wait()
                off += n

        def out_wait(b):
            pltpu.make_async_copy(
                outs[b], out_hbm.at[pl.ds(0, SROWS)], osems[b]).wait()

        def stage(c0, b):
            drain(b)
            _compute_chunk(c0 + b, iidx_v, sidx_v,
                           irows[b], srows[b], outs[b])

            @pl.when(c0 + b + 2 < NCHUNK)
            def _():
                fire(c0 + b + 2, b)
            obase = pl.multiple_of(wbase * NSAMP + (c0 + b) * SROWS, SROWS)
            pltpu.make_async_copy(
                outs[b], out_hbm.at[pl.ds(obase, SROWS)], osems[b]).start()

        fire(0, 0)
        fire(1, 1)

        def pair_body(kk, carry):
            c0 = kk * 2
            for b in range(2):
                @pl.when(kk > 0)
                def _(b=b):
                    out_wait(b)
                stage(c0, b)
            return carry

        lax.fori_loop(0, NCHUNK // 2, pair_body, 0)
        out_wait(0)
        out_wait(1)

    return k(items, samples_flat, emb2)


@jax.jit
def kernel(items, samples, embeddings):
    scores_flat = _item2vec_sc(
        items, samples.reshape(-1), embeddings.reshape(VOCAB // 2, 2 * DIM))
    return scores_flat.reshape(BATCH, NSAMP)


# speedup vs baseline: 8.4043x; 1.3574x over previous
import functools

import jax
import jax.numpy as jnp
from jax import lax
from jax.experimental import pallas as pl
from jax.experimental.pallas import tpu as pltpu
from jax.experimental.pallas import tpu_sc as plsc

VOCAB = 1000000
DIM = 64
BATCH = 16384
NSAMP = 20

W = 4096
GRID = (VOCAB + W - 1) // W
PROWS = GRID * W // 2

NUM_CORES = 2
NUM_SUBCORES = 16
NW = NUM_CORES * NUM_SUBCORES
B_PER_W = BATCH // NW
CHUNK = 16
NCHUNK = B_PER_W // CHUNK
SROWS = CHUNK * NSAMP
SSTREAMS = (128, 128, 64)
NS_ALL = B_PER_W * NSAMP
LHALF = NSAMP // 2


def _repack_tc(emb_t):
    def body(x_ref, o_ref):
        x = x_ref[...]
        o_ref[:, 0:DIM] = x[:, 0:W // 2].T
        o_ref[:, DIM:128] = x[:, W // 2:W].T

    return pl.pallas_call(
        body,
        grid=(GRID,),
        in_specs=[pl.BlockSpec((DIM, W), lambda g: (0, g))],
        out_specs=pl.BlockSpec((W // 2, 128), lambda g: (g, 0)),
        out_shape=jax.ShapeDtypeStruct((PROWS, 128), jnp.float32),
    )(emb_t)


def _packed_col64(v):
    return ((v >> 11) & 1) << 6


def _compute_chunk(c, iidx_v, sidx_v, item_rows, samp_rows, out_v):
    iota = lax.iota(jnp.int32, 16)
    ibase = pl.multiple_of(c * CHUNK, CHUNK)
    sbase = pl.multiple_of(c * SROWS, SROWS)
    item_v = iidx_v[pl.ds(ibase, 16)]
    ic64 = _packed_col64(item_v)

    for h in range(2):
        srow = [iota * NSAMP + (h * LHALF + l) for l in range(LHALF)]
        sc64 = [
            _packed_col64(plsc.load_gather(sidx_v, [sbase + srow[l]]))
            for l in range(LHALF)
        ]

        def dbody(i, acc, srow=srow, sc64=sc64):
            new_acc = list(acc)
            for u in range(2):
                d = i * 2 + u
                dcol = jnp.full((16,), d, dtype=jnp.int32)
                item_col = plsc.load_gather(item_rows, [iota, ic64 + dcol])
                for l in range(LHALF):
                    scol = plsc.load_gather(
                        samp_rows, [srow[l], sc64[l] + dcol])
                    new_acc[l] = new_acc[l] + item_col * scol
            return tuple(new_acc)

        acc = lax.fori_loop(
            0, DIM // 2, dbody,
            tuple(jnp.zeros((16,), jnp.float32) for _ in range(LHALF)))
        for l in range(LHALF):
            plsc.store_scatter(out_v, [srow[l]], acc[l])


def _item2vec_sc(items, samples_flat, packed):
    mesh = plsc.VectorSubcoreMesh(
        core_axis_name="c", subcore_axis_name="s",
        num_cores=NUM_CORES, num_subcores=NUM_SUBCORES)

    @functools.partial(
        pl.kernel,
        out_type=jax.ShapeDtypeStruct((BATCH * NSAMP,), jnp.float32),
        mesh=mesh,
        scratch_types=[
            pltpu.VMEM((B_PER_W,), jnp.int32),
            pltpu.VMEM((NS_ALL,), jnp.int32),
            pltpu.VMEM((B_PER_W,), jnp.int32),
            pltpu.VMEM((NS_ALL,), jnp.int32),
            [pltpu.VMEM((CHUNK, 128), jnp.float32)] * 2,
            [pltpu.VMEM((SROWS, 128), jnp.float32)] * 2,
            [pltpu.VMEM((SROWS,), jnp.float32)] * 2,
            [pltpu.SemaphoreType.DMA] * 2,
            [pltpu.SemaphoreType.DMA] * 2,
        ],
        compiler_params=pltpu.CompilerParams(
            needs_layout_passes=False, use_tc_tiling_on_sc=True),
    )
    def k(items_hbm, samples_hbm, emb_hbm, out_hbm,
          iidx_v, sidx_v, ipix_v, spix_v, irows, srows, outs, gsems, osems):
        wid = lax.axis_index("s") * NUM_CORES + lax.axis_index("c")
        wbase = wid * B_PER_W
        pltpu.sync_copy(items_hbm.at[pl.ds(wbase, B_PER_W)], iidx_v)
        pltpu.sync_copy(
            samples_hbm.at[pl.ds(wbase * NSAMP, NS_ALL)], sidx_v)

        def packrow(i, _, src, dst):
            off = pl.multiple_of(i * 16, 16)
            v = src[pl.ds(off, 16)]
            dst[pl.ds(off, 16)] = ((v >> 12) << 11) | (v & 2047)
            return _

        lax.fori_loop(0, B_PER_W // 16,
                      functools.partial(packrow, src=iidx_v, dst=ipix_v), 0)
        lax.fori_loop(0, NS_ALL // 16,
                      functools.partial(packrow, src=sidx_v, dst=spix_v), 0)

        def fire(c, b):
            ibase = pl.multiple_of(c * CHUNK, CHUNK)
            sbase = pl.multiple_of(c * SROWS, SROWS)
            pltpu.make_async_copy(
                emb_hbm.at[ipix_v.at[pl.ds(ibase, CHUNK)]],
                irows[b], gsems[b]).start()
            off = 0
            for n in SSTREAMS:
                pltpu.make_async_copy(
                    emb_hbm.at[spix_v.at[pl.ds(sbase + off, n)]],
                    srows[b].at[pl.ds(off, n)], gsems[b]).start()
                off += n

        def drain(b):
            pltpu.make_async_copy(
                emb_hbm.at[ipix_v.at[pl.ds(0, CHUNK)]],
                irows[b], gsems[b]).wait()
            off = 0
            for n in SSTREAMS:
                pltpu.make_async_copy(
                    emb_hbm.at[spix_v.at[pl.ds(0, n)]],
                    srows[b].at[pl.ds(off, n)], gsems[b]).wait()
                off += n

        def out_wait(b):
            pltpu.make_async_copy(
                outs[b], out_hbm.at[pl.ds(0, SROWS)], osems[b]).wait()

        def stage(c0, b):
            drain(b)
            _compute_chunk(c0 + b, iidx_v, sidx_v,
                           irows[b], srows[b], outs[b])

            @pl.when(c0 + b + 2 < NCHUNK)
            def _():
                fire(c0 + b + 2, b)
            obase = pl.multiple_of(wbase * NSAMP + (c0 + b) * SROWS, SROWS)
            pltpu.make_async_copy(
                outs[b], out_hbm.at[pl.ds(obase, SROWS)], osems[b]).start()

        fire(0, 0)
        fire(1, 1)

        def pair_body(kk, carry):
            c0 = kk * 2
            for b in range(2):
                @pl.when(kk > 0)
                def _(b=b):
                    out_wait(b)
                stage(c0, b)
            return carry

        lax.fori_loop(0, NCHUNK // 2, pair_body, 0)
        out_wait(0)
        out_wait(1)

    return k(items, samples_flat, packed)


@jax.jit
def kernel(items, samples, embeddings):
    packed = _repack_tc(embeddings.T)
    scores_flat = _item2vec_sc(items, samples.reshape(-1), packed)
    return scores_flat.reshape(BATCH, NSAMP)


# speedup vs baseline: 14.4701x; 1.7217x over previous
import functools

import jax
import jax.numpy as jnp
from jax import lax
from jax.experimental import pallas as pl
from jax.experimental.pallas import tpu as pltpu
from jax.experimental.pallas import tpu_sc as plsc

VOCAB = 1000000
DIM = 64
BATCH = 16384
NSAMP = 20

W = 4096
GRID = (VOCAB + W - 1) // W
PROWS = GRID * W // 2

NUM_CORES = 2
NUM_SUBCORES = 16
NW = NUM_CORES * NUM_SUBCORES
B_PER_W = BATCH // NW
CHUNK = 16
NCHUNK = B_PER_W // CHUNK
SROWS = CHUNK * NSAMP
SSTREAMS = (128, 128, 64)
NS_ALL = B_PER_W * NSAMP
LHALF = NSAMP // 2


def _repack_tc(emb_t):
    def body(x_ref, o_ref):
        x = x_ref[...]
        o_ref[:, 0:DIM] = x[:, 0:W // 2].T
        o_ref[:, DIM:128] = x[:, W // 2:W].T

    return pl.pallas_call(
        body,
        grid=(GRID,),
        in_specs=[pl.BlockSpec((DIM, W), lambda g: (0, g))],
        out_specs=pl.BlockSpec((W // 2, 128), lambda g: (g, 0)),
        out_shape=jax.ShapeDtypeStruct((PROWS, 128), jnp.float32),
    )(emb_t)


def _compute_chunk(c, iidx_v, sidx_v, item_rows, samp_rows, out_v):
    iota = lax.iota(jnp.int32, 16)

    def bbody(b, carry):
        its = [item_rows[b, pl.ds(16 * k, 16)] for k in range(4)]
        srow0 = b * NSAMP
        vecs = [jnp.zeros((16,), jnp.float32), jnp.zeros((16,), jnp.float32)]
        for l in range(NSAMP):
            srow = srow0 + l
            p = its[0] * samp_rows[srow, pl.ds(0, 16)]
            for k in range(1, 4):
                p = p + its[k] * samp_rows[srow, pl.ds(16 * k, 16)]
            sc = jnp.sum(p)
            vecs[l // 16] = jnp.where(
                iota == (l % 16), jnp.full((16,), sc), vecs[l // 16])
        plsc.store_scatter(out_v, [srow0 + iota], vecs[0])
        plsc.store_scatter(out_v, [srow0 + 16 + iota], vecs[1],
                           mask=iota < 4)
        return carry

    lax.fori_loop(0, CHUNK, bbody, 0)


def _item2vec_sc(items, samples_flat, packed):
    mesh = plsc.VectorSubcoreMesh(
        core_axis_name="c", subcore_axis_name="s",
        num_cores=NUM_CORES, num_subcores=NUM_SUBCORES)

    @functools.partial(
        pl.kernel,
        out_type=jax.ShapeDtypeStruct((BATCH * NSAMP,), jnp.float32),
        mesh=mesh,
        scratch_types=[
            pltpu.VMEM((B_PER_W,), jnp.int32),
            pltpu.VMEM((NS_ALL,), jnp.int32),
            pltpu.VMEM((B_PER_W,), jnp.int32),
            pltpu.VMEM((NS_ALL,), jnp.int32),
            [pltpu.VMEM((CHUNK, DIM), jnp.float32)] * 2,
            [pltpu.VMEM((SROWS, DIM), jnp.float32)] * 2,
            [pltpu.VMEM((SROWS,), jnp.float32)] * 2,
            [pltpu.SemaphoreType.DMA] * 2,
            [pltpu.SemaphoreType.DMA] * 2,
        ],
        compiler_params=pltpu.CompilerParams(
            needs_layout_passes=False, use_tc_tiling_on_sc=False),
    )
    def k(items_hbm, samples_hbm, emb_hbm, out_hbm,
          iidx_v, sidx_v, ipix_v, spix_v, irows, srows, outs, gsems, osems):
        wid = lax.axis_index("s") * NUM_CORES + lax.axis_index("c")
        wbase = wid * B_PER_W
        pltpu.sync_copy(items_hbm.at[pl.ds(wbase, B_PER_W)], iidx_v)
        pltpu.sync_copy(
            samples_hbm.at[pl.ds(wbase * NSAMP, NS_ALL)], sidx_v)

        def packrow(i, _, src, dst):
            off = pl.multiple_of(i * 16, 16)
            v = src[pl.ds(off, 16)]
            dst[pl.ds(off, 16)] = (
                ((v >> 12) << 12) | ((v & 2047) << 1) | ((v >> 11) & 1))
            return _

        lax.fori_loop(0, B_PER_W // 16,
                      functools.partial(packrow, src=iidx_v, dst=ipix_v), 0)
        lax.fori_loop(0, NS_ALL // 16,
                      functools.partial(packrow, src=sidx_v, dst=spix_v), 0)

        def fire(c, b):
            ibase = pl.multiple_of(c * CHUNK, CHUNK)
            sbase = pl.multiple_of(c * SROWS, SROWS)
            pltpu.make_async_copy(
                emb_hbm.at[ipix_v.at[pl.ds(ibase, CHUNK)]],
                irows[b], gsems[b]).start()
            off = 0
            for n in SSTREAMS:
                pltpu.make_async_copy(
                    emb_hbm.at[spix_v.at[pl.ds(sbase + off, n)]],
                    srows[b].at[pl.ds(off, n)], gsems[b]).start()
                off += n

        def drain(b):
            pltpu.make_async_copy(
                emb_hbm.at[ipix_v.at[pl.ds(0, CHUNK)]],
                irows[b], gsems[b]).wait()
            off = 0
            for n in SSTREAMS:
                pltpu.make_async_copy(
                    emb_hbm.at[spix_v.at[pl.ds(0, n)]],
                    srows[b].at[pl.ds(off, n)], gsems[b]).wait()
                off += n

        def out_wait(b):
            pltpu.make_async_copy(
                outs[b], out_hbm.at[pl.ds(0, SROWS)], osems[b]).wait()

        def stage(c0, b):
            with jax.named_scope("drain"):
                drain(b)
            with jax.named_scope("compute"):
                _compute_chunk(c0 + b, iidx_v, sidx_v,
                               irows[b], srows[b], outs[b])

            @pl.when(c0 + b + 2 < NCHUNK)
            def _():
                fire(c0 + b + 2, b)
            obase = pl.multiple_of(wbase * NSAMP + (c0 + b) * SROWS, SROWS)
            pltpu.make_async_copy(
                outs[b], out_hbm.at[pl.ds(obase, SROWS)], osems[b]).start()

        fire(0, 0)
        fire(1, 1)

        def pair_body(kk, carry):
            c0 = kk * 2
            for b in range(2):
                @pl.when(kk > 0)
                def _(b=b):
                    out_wait(b)
                stage(c0, b)
            return carry

        lax.fori_loop(0, NCHUNK // 2, pair_body, 0)
        out_wait(0)
        out_wait(1)

    return k(items, samples_flat, packed)


@jax.jit
def kernel(items, samples, embeddings):
    packed = _repack_tc(embeddings.T).reshape(2 * PROWS, DIM)
    scores_flat = _item2vec_sc(items, samples.reshape(-1), packed)
    return scores_flat.reshape(BATCH, NSAMP)


# speedup vs baseline: 14.9173x; 1.0309x over previous
import functools

import jax
import jax.numpy as jnp
from jax import lax
from jax.experimental import pallas as pl
from jax.experimental.pallas import tpu as pltpu
from jax.experimental.pallas import tpu_sc as plsc

VOCAB = 1000000
DIM = 64
BATCH = 16384
NSAMP = 20

W = 4096
GRID = (VOCAB + W - 1) // W
PROWS = GRID * W // 2

NUM_CORES = 2
NUM_SUBCORES = 16
NW = NUM_CORES * NUM_SUBCORES
B_PER_W = BATCH // NW
CHUNK = 16
NCHUNK = B_PER_W // CHUNK
SROWS = CHUNK * NSAMP
SSTREAMS = (128, 128, 64)
NS_ALL = B_PER_W * NSAMP
LHALF = NSAMP // 2


def _repack_tc(emb_t):
    def body(x_ref, o_ref):
        eye = (lax.broadcasted_iota(jnp.int32, (DIM, DIM), 0)
               == lax.broadcasted_iota(jnp.int32, (DIM, DIM), 1)
               ).astype(jnp.float32)
        y = lax.dot_general(
            x_ref[...], eye, (((0,), (0,)), ((), ())),
            preferred_element_type=jnp.float32)
        o_ref[:, 0:DIM] = y[0:W // 2, :]
        o_ref[:, DIM:128] = y[W // 2:W, :]

    return pl.pallas_call(
        body,
        grid=(GRID,),
        in_specs=[pl.BlockSpec((DIM, W), lambda g: (0, g))],
        out_specs=pl.BlockSpec((W // 2, 128), lambda g: (g, 0)),
        out_shape=jax.ShapeDtypeStruct((PROWS, 128), jnp.float32),
    )(emb_t)


def _compute_chunk(c, iidx_v, sidx_v, item_rows, samp_rows, out_v):
    iota = lax.iota(jnp.int32, 16)

    gb0 = pl.multiple_of(c * CHUNK, CHUNK)

    def bbody(b, carry):
        its = [item_rows[b, pl.ds(16 * k, 16)] for k in range(4)]
        srow0 = b * NSAMP
        gb = gb0 + b
        vecs = [jnp.zeros((16,), jnp.float32), jnp.zeros((16,), jnp.float32)]
        for l in range(NSAMP):
            srow = srow0 + l
            p = its[0] * samp_rows[srow, pl.ds(0, 16)]
            for k in range(1, 4):
                p = p + its[k] * samp_rows[srow, pl.ds(16 * k, 16)]
            sc = jnp.sum(p)
            vecs[l // 16] = jnp.where(
                iota == (l % 16), jnp.full((16,), sc), vecs[l // 16])
        plsc.store_scatter(out_v, [iota * B_PER_W + gb], vecs[0])
        plsc.store_scatter(out_v, [(16 + iota) * B_PER_W + gb], vecs[1],
                           mask=iota < 4)
        return carry

    lax.fori_loop(0, CHUNK, bbody, 0)


def _item2vec_sc(items, samples_flat, packed):
    mesh = plsc.VectorSubcoreMesh(
        core_axis_name="c", subcore_axis_name="s",
        num_cores=NUM_CORES, num_subcores=NUM_SUBCORES)

    @functools.partial(
        pl.kernel,
        out_type=jax.ShapeDtypeStruct((BATCH * NSAMP,), jnp.float32),
        mesh=mesh,
        scratch_types=[
            pltpu.VMEM((B_PER_W,), jnp.int32),
            pltpu.VMEM((NS_ALL,), jnp.int32),
            pltpu.VMEM((B_PER_W,), jnp.int32),
            pltpu.VMEM((NS_ALL,), jnp.int32),
            [pltpu.VMEM((CHUNK, DIM), jnp.float32)] * 2,
            [pltpu.VMEM((SROWS, DIM), jnp.float32)] * 2,
            pltpu.VMEM((NS_ALL,), jnp.float32),
            [pltpu.SemaphoreType.DMA] * 2,
            pltpu.SemaphoreType.DMA,
        ],
        compiler_params=pltpu.CompilerParams(
            needs_layout_passes=False, use_tc_tiling_on_sc=False),
    )
    def k(items_hbm, samples_hbm, emb_hbm, out_hbm,
          iidx_v, sidx_v, ipix_v, spix_v, irows, srows, obuf, gsems, osem):
        wid = lax.axis_index("s") * NUM_CORES + lax.axis_index("c")
        wbase = wid * B_PER_W
        pltpu.sync_copy(items_hbm.at[pl.ds(wbase, B_PER_W)], iidx_v)
        pltpu.sync_copy(
            samples_hbm.at[pl.ds(wbase * NSAMP, NS_ALL)], sidx_v)

        def packrow(i, _, src, dst):
            off = pl.multiple_of(i * 16, 16)
            v = src[pl.ds(off, 16)]
            dst[pl.ds(off, 16)] = (
                ((v >> 12) << 12) | ((v & 2047) << 1) | ((v >> 11) & 1))
            return _

        lax.fori_loop(0, B_PER_W // 16,
                      functools.partial(packrow, src=iidx_v, dst=ipix_v), 0)
        lax.fori_loop(0, NS_ALL // 16,
                      functools.partial(packrow, src=sidx_v, dst=spix_v), 0)

        def fire(c, b):
            ibase = pl.multiple_of(c * CHUNK, CHUNK)
            sbase = pl.multiple_of(c * SROWS, SROWS)
            pltpu.make_async_copy(
                emb_hbm.at[ipix_v.at[pl.ds(ibase, CHUNK)]],
                irows[b], gsems[b]).start()
            off = 0
            for n in SSTREAMS:
                pltpu.make_async_copy(
                    emb_hbm.at[spix_v.at[pl.ds(sbase + off, n)]],
                    srows[b].at[pl.ds(off, n)], gsems[b]).start()
                off += n

        def drain(b):
            pltpu.make_async_copy(
                emb_hbm.at[ipix_v.at[pl.ds(0, CHUNK)]],
                irows[b], gsems[b]).wait()
            off = 0
            for n in SSTREAMS:
                pltpu.make_async_copy(
                    emb_hbm.at[spix_v.at[pl.ds(0, n)]],
                    srows[b].at[pl.ds(off, n)], gsems[b]).wait()
                off += n

        def stage(c0, b):
            drain(b)
            _compute_chunk(c0 + b, iidx_v, sidx_v,
                           irows[b], srows[b], obuf)

            @pl.when(c0 + b + 2 < NCHUNK)
            def _():
                fire(c0 + b + 2, b)

        fire(0, 0)
        fire(1, 1)

        def pair_body(kk, carry):
            c0 = kk * 2
            for b in range(2):
                stage(c0, b)
            return carry

        lax.fori_loop(0, NCHUNK // 2, pair_body, 0)
        for l in range(NSAMP):
            pltpu.make_async_copy(
                obuf.at[pl.ds(l * B_PER_W, B_PER_W)],
                out_hbm.at[pl.ds(l * BATCH + wbase, B_PER_W)],
                osem).start()
        for l in range(NSAMP):
            pltpu.make_async_copy(
                obuf.at[pl.ds(0, B_PER_W)],
                out_hbm.at[pl.ds(wbase, B_PER_W)], osem).wait()

    return k(items, samples_flat, packed)


@jax.jit
def kernel(items, samples, embeddings):
    packed = _repack_tc(embeddings.T).reshape(2 * PROWS, DIM)
    scores_flat = _item2vec_sc(items, samples.reshape(-1), packed)
    return scores_flat.reshape(NSAMP, BATCH).T


# speedup vs baseline: 19.8147x; 1.3283x over previous
import functools

import jax
import jax.numpy as jnp
from jax import lax
from jax.experimental import pallas as pl
from jax.experimental.pallas import tpu as pltpu
from jax.experimental.pallas import tpu_sc as plsc

VOCAB = 1000000
DIM = 64
BATCH = 16384
NSAMP = 20

W = 16384
LOGW = 14
GRID = (VOCAB + W - 1) // W
PROWS = GRID * W // 2

NUM_CORES = 2
NUM_SUBCORES = 16
NW = NUM_CORES * NUM_SUBCORES
B_PER_W = BATCH // NW
CHUNK = 16
NCHUNK = B_PER_W // CHUNK
SROWS = CHUNK * NSAMP
SSTREAMS = (128, 128, 64)
NS_ALL = B_PER_W * NSAMP
LHALF = NSAMP // 2


def _repack_tc(emb_t):
    def body(x_ref, o_ref):
        eye = (lax.broadcasted_iota(jnp.int32, (DIM, DIM), 0)
               == lax.broadcasted_iota(jnp.int32, (DIM, DIM), 1)
               ).astype(jnp.float32)
        y = lax.dot_general(
            x_ref[...], eye, (((0,), (0,)), ((), ())),
            preferred_element_type=jnp.float32)
        o_ref[:, 0:DIM] = y[0:W // 2, :]
        o_ref[:, DIM:128] = y[W // 2:W, :]

    return pl.pallas_call(
        body,
        grid=(GRID,),
        in_specs=[pl.BlockSpec((DIM, W), lambda g: (0, g))],
        out_specs=pl.BlockSpec((W // 2, 128), lambda g: (g, 0)),
        out_shape=jax.ShapeDtypeStruct((PROWS, 128), jnp.float32),
    )(emb_t)


def _compute_chunk(c, iidx_v, sidx_v, item_rows, samp_rows, out_v):
    iota = lax.iota(jnp.int32, 16)

    gb0 = pl.multiple_of(c * CHUNK, CHUNK)

    def bbody(b, carry):
        its = [item_rows[b, pl.ds(16 * k, 16)] for k in range(4)]
        srow0 = b * NSAMP
        gb = gb0 + b
        vecs = [jnp.zeros((16,), jnp.float32), jnp.zeros((16,), jnp.float32)]
        for l in range(NSAMP):
            srow = srow0 + l
            p = its[0] * samp_rows[srow, pl.ds(0, 16)]
            for k in range(1, 4):
                p = p + its[k] * samp_rows[srow, pl.ds(16 * k, 16)]
            sc = jnp.sum(p)
            vecs[l // 16] = jnp.where(
                iota == (l % 16), jnp.full((16,), sc), vecs[l // 16])
        plsc.store_scatter(out_v, [iota * B_PER_W + gb], vecs[0])
        plsc.store_scatter(out_v, [(16 + iota) * B_PER_W + gb], vecs[1],
                           mask=iota < 4)
        return carry

    lax.fori_loop(0, CHUNK, bbody, 0)


def _item2vec_sc(items, samples_flat, packed):
    mesh = plsc.VectorSubcoreMesh(
        core_axis_name="c", subcore_axis_name="s",
        num_cores=NUM_CORES, num_subcores=NUM_SUBCORES)

    @functools.partial(
        pl.kernel,
        out_type=jax.ShapeDtypeStruct((BATCH * NSAMP,), jnp.float32),
        mesh=mesh,
        scratch_types=[
            pltpu.VMEM((B_PER_W,), jnp.int32),
            pltpu.VMEM((NS_ALL,), jnp.int32),
            pltpu.VMEM((B_PER_W,), jnp.int32),
            pltpu.VMEM((NS_ALL,), jnp.int32),
            [pltpu.VMEM((CHUNK, DIM), jnp.float32)] * 2,
            [pltpu.VMEM((SROWS, DIM), jnp.float32)] * 2,
            pltpu.VMEM((NS_ALL,), jnp.float32),
            [pltpu.SemaphoreType.DMA] * 2,
            pltpu.SemaphoreType.DMA,
        ],
        compiler_params=pltpu.CompilerParams(
            needs_layout_passes=False, use_tc_tiling_on_sc=False),
    )
    def k(items_hbm, samples_hbm, emb_hbm, out_hbm,
          iidx_v, sidx_v, ipix_v, spix_v, irows, srows, obuf, gsems, osem):
        wid = lax.axis_index("s") * NUM_CORES + lax.axis_index("c")
        wbase = wid * B_PER_W
        pltpu.sync_copy(items_hbm.at[pl.ds(wbase, B_PER_W)], iidx_v)
        pltpu.sync_copy(
            samples_hbm.at[pl.ds(wbase * NSAMP, NS_ALL)], sidx_v)

        def packrow(i, _, src, dst):
            off = pl.multiple_of(i * 16, 16)
            v = src[pl.ds(off, 16)]
            dst[pl.ds(off, 16)] = (
                ((v >> LOGW) << LOGW)
                | ((v & (W // 2 - 1)) << 1)
                | ((v >> (LOGW - 1)) & 1))
            return _

        lax.fori_loop(0, B_PER_W // 16,
                      functools.partial(packrow, src=iidx_v, dst=ipix_v), 0)
        lax.fori_loop(0, NS_ALL // 16,
                      functools.partial(packrow, src=sidx_v, dst=spix_v), 0)

        def fire(c, b):
            ibase = pl.multiple_of(c * CHUNK, CHUNK)
            sbase = pl.multiple_of(c * SROWS, SROWS)
            pltpu.make_async_copy(
                emb_hbm.at[ipix_v.at[pl.ds(ibase, CHUNK)]],
                irows[b], gsems[b]).start()
            off = 0
            for n in SSTREAMS:
                pltpu.make_async_copy(
                    emb_hbm.at[spix_v.at[pl.ds(sbase + off, n)]],
                    srows[b].at[pl.ds(off, n)], gsems[b]).start()
                off += n

        def drain(b):
            pltpu.make_async_copy(
                emb_hbm.at[ipix_v.at[pl.ds(0, CHUNK)]],
                irows[b], gsems[b]).wait()
            off = 0
            for n in SSTREAMS:
                pltpu.make_async_copy(
                    emb_hbm.at[spix_v.at[pl.ds(0, n)]],
                    srows[b].at[pl.ds(off, n)], gsems[b]).wait()
                off += n

        def stage(c0, b):
            drain(b)
            _compute_chunk(c0 + b, iidx_v, sidx_v,
                           irows[b], srows[b], obuf)

            @pl.when(c0 + b + 2 < NCHUNK)
            def _():
                fire(c0 + b + 2, b)

        fire(0, 0)
        fire(1, 1)

        def pair_body(kk, carry):
            c0 = kk * 2
            for b in range(2):
                stage(c0, b)
            return carry

        lax.fori_loop(0, NCHUNK // 2, pair_body, 0)
        for l in range(NSAMP):
            pltpu.make_async_copy(
                obuf.at[pl.ds(l * B_PER_W, B_PER_W)],
                out_hbm.at[pl.ds(l * BATCH + wbase, B_PER_W)],
                osem).start()
        for l in range(NSAMP):
            pltpu.make_async_copy(
                obuf.at[pl.ds(0, B_PER_W)],
                out_hbm.at[pl.ds(wbase, B_PER_W)], osem).wait()

    return k(items, samples_flat, packed)


@jax.jit
def kernel(items, samples, embeddings):
    packed = _repack_tc(embeddings.T).reshape(2 * PROWS, DIM)
    scores_flat = _item2vec_sc(items, samples.reshape(-1), packed)
    return scores_flat.reshape(NSAMP, BATCH).T


# speedup vs baseline: 20.7348x; 1.0464x over previous
import functools

import jax
import jax.numpy as jnp
from jax import lax
from jax.experimental import pallas as pl
from jax.experimental.pallas import tpu as pltpu
from jax.experimental.pallas import tpu_sc as plsc

VOCAB = 1000000
DIM = 64
BATCH = 16384
NSAMP = 20

W = 32768
LOGW = 15
GRID = (VOCAB + W - 1) // W
PROWS = GRID * W // 2

NUM_CORES = 2
NUM_SUBCORES = 16
NW = NUM_CORES * NUM_SUBCORES
B_PER_W = BATCH // NW
CHUNK = 16
NCHUNK = B_PER_W // CHUNK
SROWS = CHUNK * NSAMP
SSTREAMS = (128, 128, 64)
NS_ALL = B_PER_W * NSAMP
LHALF = NSAMP // 2


def _repack_tc(emb_t):
    def body(x_ref, o_ref):
        eye = (lax.broadcasted_iota(jnp.int32, (DIM, DIM), 0)
               == lax.broadcasted_iota(jnp.int32, (DIM, DIM), 1)
               ).astype(jnp.float32)
        y = lax.dot_general(
            x_ref[...], eye, (((0,), (0,)), ((), ())),
            preferred_element_type=jnp.float32)
        o_ref[:, 0:DIM] = y[0:W // 2, :]
        o_ref[:, DIM:128] = y[W // 2:W, :]

    return pl.pallas_call(
        body,
        grid=(GRID,),
        in_specs=[pl.BlockSpec((DIM, W), lambda g: (0, g))],
        out_specs=pl.BlockSpec((W // 2, 128), lambda g: (g, 0)),
        out_shape=jax.ShapeDtypeStruct((PROWS, 128), jnp.float32),
    )(emb_t)


def _compute_chunk(c, iidx_v, sidx_v, item_rows, samp_rows, out_v):
    iota = lax.iota(jnp.int32, 16)

    gb0 = pl.multiple_of(c * CHUNK, CHUNK)

    def bbody(b, carry):
        its = [item_rows[b, pl.ds(16 * k, 16)] for k in range(4)]
        srow0 = b * NSAMP
        gb = gb0 + b
        vecs = [jnp.zeros((16,), jnp.float32), jnp.zeros((16,), jnp.float32)]
        for l in range(NSAMP):
            srow = srow0 + l
            p = its[0] * samp_rows[srow, pl.ds(0, 16)]
            for k in range(1, 4):
                p = p + its[k] * samp_rows[srow, pl.ds(16 * k, 16)]
            sc = jnp.sum(p)
            vecs[l // 16] = jnp.where(
                iota == (l % 16), jnp.full((16,), sc), vecs[l // 16])
        plsc.store_scatter(out_v, [iota * B_PER_W + gb], vecs[0])
        plsc.store_scatter(out_v, [(16 + iota) * B_PER_W + gb], vecs[1],
                           mask=iota < 4)
        return carry

    lax.fori_loop(0, CHUNK, bbody, 0)


def _item2vec_sc(items, samples_flat, packed):
    mesh = plsc.VectorSubcoreMesh(
        core_axis_name="c", subcore_axis_name="s",
        num_cores=NUM_CORES, num_subcores=NUM_SUBCORES)

    @functools.partial(
        pl.kernel,
        out_type=jax.ShapeDtypeStruct((BATCH * NSAMP,), jnp.float32),
        mesh=mesh,
        scratch_types=[
            pltpu.VMEM((B_PER_W,), jnp.int32),
            pltpu.VMEM((NS_ALL,), jnp.int32),
            pltpu.VMEM((B_PER_W,), jnp.int32),
            pltpu.VMEM((NS_ALL,), jnp.int32),
            [pltpu.VMEM((CHUNK, DIM), jnp.float32)] * 2,
            [pltpu.VMEM((SROWS, DIM), jnp.float32)] * 2,
            pltpu.VMEM((NS_ALL,), jnp.float32),
            [pltpu.SemaphoreType.DMA] * 2,
            pltpu.SemaphoreType.DMA,
        ],
        compiler_params=pltpu.CompilerParams(
            needs_layout_passes=False, use_tc_tiling_on_sc=False),
    )
    def k(items_hbm, samples_hbm, emb_hbm, out_hbm,
          iidx_v, sidx_v, ipix_v, spix_v, irows, srows, obuf, gsems, osem):
        wid = lax.axis_index("s") * NUM_CORES + lax.axis_index("c")
        wbase = wid * B_PER_W
        pltpu.sync_copy(items_hbm.at[pl.ds(wbase, B_PER_W)], iidx_v)
        pltpu.sync_copy(
            samples_hbm.at[pl.ds(wbase * NSAMP, NS_ALL)], sidx_v)

        def packrow(i, _, src, dst):
            off = pl.multiple_of(i * 16, 16)
            v = src[pl.ds(off, 16)]
            dst[pl.ds(off, 16)] = (
                ((v >> LOGW) << LOGW)
                | ((v & (W // 2 - 1)) << 1)
                | ((v >> (LOGW - 1)) & 1))
            return _

        lax.fori_loop(0, B_PER_W // 16,
                      functools.partial(packrow, src=iidx_v, dst=ipix_v), 0)
        lax.fori_loop(0, NS_ALL // 16,
                      functools.partial(packrow, src=sidx_v, dst=spix_v), 0)

        def fire(c, b):
            ibase = pl.multiple_of(c * CHUNK, CHUNK)
            sbase = pl.multiple_of(c * SROWS, SROWS)
            pltpu.make_async_copy(
                emb_hbm.at[ipix_v.at[pl.ds(ibase, CHUNK)]],
                irows[b], gsems[b]).start()
            off = 0
            for n in SSTREAMS:
                pltpu.make_async_copy(
                    emb_hbm.at[spix_v.at[pl.ds(sbase + off, n)]],
                    srows[b].at[pl.ds(off, n)], gsems[b]).start()
                off += n

        def drain(b):
            pltpu.make_async_copy(
                emb_hbm.at[ipix_v.at[pl.ds(0, CHUNK)]],
                irows[b], gsems[b]).wait()
            off = 0
            for n in SSTREAMS:
                pltpu.make_async_copy(
                    emb_hbm.at[spix_v.at[pl.ds(0, n)]],
                    srows[b].at[pl.ds(off, n)], gsems[b]).wait()
                off += n

        def stage(c0, b):
            drain(b)
            _compute_chunk(c0 + b, iidx_v, sidx_v,
                           irows[b], srows[b], obuf)

            @pl.when(c0 + b + 2 < NCHUNK)
            def _():
                fire(c0 + b + 2, b)

        fire(0, 0)
        fire(1, 1)

        def pair_body(kk, carry):
            c0 = kk * 2
            for b in range(2):
                stage(c0, b)
            return carry

        lax.fori_loop(0, NCHUNK // 2, pair_body, 0)
        for l in range(NSAMP):
            pltpu.make_async_copy(
                obuf.at[pl.ds(l * B_PER_W, B_PER_W)],
                out_hbm.at[pl.ds(l * BATCH + wbase, B_PER_W)],
                osem).start()
        for l in range(NSAMP):
            pltpu.make_async_copy(
                obuf.at[pl.ds(0, B_PER_W)],
                out_hbm.at[pl.ds(wbase, B_PER_W)], osem).wait()

    return k(items, samples_flat, packed)


@jax.jit
def kernel(items, samples, embeddings):
    packed = _repack_tc(embeddings.T).reshape(2 * PROWS, DIM)
    scores_flat = _item2vec_sc(items, samples.reshape(-1), packed)
    return scores_flat.reshape(NSAMP, BATCH).T


# speedup vs baseline: 20.9509x; 1.0104x over previous
import functools

import jax
import jax.numpy as jnp
from jax import lax
from jax.experimental import pallas as pl
from jax.experimental.pallas import tpu as pltpu
from jax.experimental.pallas import tpu_sc as plsc

VOCAB = 1000000
DIM = 64
BATCH = 16384
NSAMP = 20

W = 32768
LOGW = 15
GRID = (VOCAB + W - 1) // W
PROWS = GRID * W // 2

NUM_CORES = 2
NUM_SUBCORES = 16
NW = NUM_CORES * NUM_SUBCORES
B_PER_W = BATCH // NW
CHUNK = 32
NCHUNK = B_PER_W // CHUNK
SROWS = CHUNK * NSAMP
SSTREAMS = (128,) * 5
NS_ALL = B_PER_W * NSAMP
LHALF = NSAMP // 2


def _repack_tc(emb_t):
    def body(x_ref, o_ref):
        eye = (lax.broadcasted_iota(jnp.int32, (DIM, DIM), 0)
               == lax.broadcasted_iota(jnp.int32, (DIM, DIM), 1)
               ).astype(jnp.float32)
        y = lax.dot_general(
            x_ref[...], eye, (((0,), (0,)), ((), ())),
            preferred_element_type=jnp.float32)
        o_ref[:, 0:DIM] = y[0:W // 2, :]
        o_ref[:, DIM:128] = y[W // 2:W, :]

    return pl.pallas_call(
        body,
        grid=(GRID,),
        in_specs=[pl.BlockSpec((DIM, W), lambda g: (0, g))],
        out_specs=pl.BlockSpec((W // 2, 128), lambda g: (g, 0)),
        out_shape=jax.ShapeDtypeStruct((PROWS, 128), jnp.float32),
    )(emb_t)


def _compute_chunk(c, iidx_v, sidx_v, item_rows, samp_rows, out_v):
    iota = lax.iota(jnp.int32, 16)

    gb0 = pl.multiple_of(c * CHUNK, CHUNK)

    def bbody(b, carry):
        its = [item_rows[b, pl.ds(16 * k, 16)] for k in range(4)]
        srow0 = b * NSAMP
        gb = gb0 + b
        vecs = [jnp.zeros((16,), jnp.float32), jnp.zeros((16,), jnp.float32)]
        for l in range(NSAMP):
            srow = srow0 + l
            p = its[0] * samp_rows[srow, pl.ds(0, 16)]
            for k in range(1, 4):
                p = p + its[k] * samp_rows[srow, pl.ds(16 * k, 16)]
            sc = jnp.sum(p)
            vecs[l // 16] = jnp.where(
                iota == (l % 16), jnp.full((16,), sc), vecs[l // 16])
        plsc.store_scatter(out_v, [iota * B_PER_W + gb], vecs[0])
        plsc.store_scatter(out_v, [(16 + iota) * B_PER_W + gb], vecs[1],
                           mask=iota < 4)
        return carry

    lax.fori_loop(0, CHUNK, bbody, 0)


def _item2vec_sc(items, samples_flat, packed):
    mesh = plsc.VectorSubcoreMesh(
        core_axis_name="c", subcore_axis_name="s",
        num_cores=NUM_CORES, num_subcores=NUM_SUBCORES)

    @functools.partial(
        pl.kernel,
        out_type=jax.ShapeDtypeStruct((BATCH * NSAMP,), jnp.float32),
        mesh=mesh,
        scratch_types=[
            pltpu.VMEM((B_PER_W,), jnp.int32),
            pltpu.VMEM((NS_ALL,), jnp.int32),
            pltpu.VMEM((B_PER_W,), jnp.int32),
            pltpu.VMEM((NS_ALL,), jnp.int32),
            [pltpu.VMEM((CHUNK, DIM), jnp.float32)] * 2,
            [pltpu.VMEM((SROWS, DIM), jnp.float32)] * 2,
            pltpu.VMEM((NS_ALL,), jnp.float32),
            [pltpu.SemaphoreType.DMA] * 2,
            pltpu.SemaphoreType.DMA,
        ],
        compiler_params=pltpu.CompilerParams(
            needs_layout_passes=False, use_tc_tiling_on_sc=False),
    )
    def k(items_hbm, samples_hbm, emb_hbm, out_hbm,
          iidx_v, sidx_v, ipix_v, spix_v, irows, srows, obuf, gsems, osem):
        wid = lax.axis_index("s") * NUM_CORES + lax.axis_index("c")
        wbase = wid * B_PER_W
        pltpu.sync_copy(items_hbm.at[pl.ds(wbase, B_PER_W)], iidx_v)
        pltpu.sync_copy(
            samples_hbm.at[pl.ds(wbase * NSAMP, NS_ALL)], sidx_v)

        def packrow(i, _, src, dst):
            off = pl.multiple_of(i * 16, 16)
            v = src[pl.ds(off, 16)]
            dst[pl.ds(off, 16)] = (
                ((v >> LOGW) << LOGW)
                | ((v & (W // 2 - 1)) << 1)
                | ((v >> (LOGW - 1)) & 1))
            return _

        lax.fori_loop(0, B_PER_W // 16,
                      functools.partial(packrow, src=iidx_v, dst=ipix_v), 0)
        lax.fori_loop(0, NS_ALL // 16,
                      functools.partial(packrow, src=sidx_v, dst=spix_v), 0)

        def fire(c, b):
            ibase = pl.multiple_of(c * CHUNK, CHUNK)
            sbase = pl.multiple_of(c * SROWS, SROWS)
            pltpu.make_async_copy(
                emb_hbm.at[ipix_v.at[pl.ds(ibase, CHUNK)]],
                irows[b], gsems[b]).start()
            off = 0
            for n in SSTREAMS:
                pltpu.make_async_copy(
                    emb_hbm.at[spix_v.at[pl.ds(sbase + off, n)]],
                    srows[b].at[pl.ds(off, n)], gsems[b]).start()
                off += n

        def drain(b):
            pltpu.make_async_copy(
                emb_hbm.at[ipix_v.at[pl.ds(0, CHUNK)]],
                irows[b], gsems[b]).wait()
            off = 0
            for n in SSTREAMS:
                pltpu.make_async_copy(
                    emb_hbm.at[spix_v.at[pl.ds(0, n)]],
                    srows[b].at[pl.ds(off, n)], gsems[b]).wait()
                off += n

        def stage(c0, b):
            drain(b)
            _compute_chunk(c0 + b, iidx_v, sidx_v,
                           irows[b], srows[b], obuf)

            @pl.when(c0 + b + 2 < NCHUNK)
            def _():
                fire(c0 + b + 2, b)

        fire(0, 0)
        fire(1, 1)

        def pair_body(kk, carry):
            c0 = kk * 2
            for b in range(2):
                stage(c0, b)
            return carry

        lax.fori_loop(0, NCHUNK // 2, pair_body, 0)
        for l in range(NSAMP):
            pltpu.make_async_copy(
                obuf.at[pl.ds(l * B_PER_W, B_PER_W)],
                out_hbm.at[pl.ds(l * BATCH + wbase, B_PER_W)],
                osem).start()
        for l in range(NSAMP):
            pltpu.make_async_copy(
                obuf.at[pl.ds(0, B_PER_W)],
                out_hbm.at[pl.ds(wbase, B_PER_W)], osem).wait()

    return k(items, samples_flat, packed)


@jax.jit
def kernel(items, samples, embeddings):
    packed = _repack_tc(embeddings.T).reshape(2 * PROWS, DIM)
    scores_flat = _item2vec_sc(items, samples.reshape(-1), packed)
    return scores_flat.reshape(NSAMP, BATCH).T


# speedup vs baseline: 25.2747x; 1.2064x over previous
import functools

import jax
import jax.numpy as jnp
from jax import lax
from jax.experimental import pallas as pl
from jax.experimental.pallas import tpu as pltpu
from jax.experimental.pallas import tpu_sc as plsc

VOCAB = 1000000
DIM = 64
BATCH = 16384
NSAMP = 20

W = 32768
LOGW = 15
GRID = (VOCAB + W - 1) // W
PROWS = GRID * W // 2

NUM_CORES = 2
NUM_SUBCORES = 16
NW = NUM_CORES * NUM_SUBCORES
B_PER_W = BATCH // NW
CHUNK = 32
NCHUNK = B_PER_W // CHUNK
SROWS = CHUNK * NSAMP
SSTREAMS = (128,) * 5
NS_ALL = B_PER_W * NSAMP
LHALF = NSAMP // 2


def _repack_tc(emb_t):
    def body(x_ref, o_ref):
        eye = (lax.broadcasted_iota(jnp.int32, (128, 128), 0)
               == lax.broadcasted_iota(jnp.int32, (128, 128), 1)
               ).astype(jnp.float32)
        xs = jnp.concatenate(
            [x_ref[:, 0:W // 2], x_ref[:, W // 2:W]], axis=0)
        o_ref[...] = lax.dot_general(
            xs, eye, (((0,), (0,)), ((), ())),
            preferred_element_type=jnp.float32)

    return pl.pallas_call(
        body,
        grid=(GRID,),
        in_specs=[pl.BlockSpec((DIM, W), lambda g: (0, g))],
        out_specs=pl.BlockSpec((W // 2, 128), lambda g: (g, 0)),
        out_shape=jax.ShapeDtypeStruct((PROWS, 128), jnp.float32),
    )(emb_t)


def _compute_chunk(c, iidx_v, sidx_v, item_rows, samp_rows, out_v):
    iota = lax.iota(jnp.int32, 16)

    gb0 = pl.multiple_of(c * CHUNK, CHUNK)

    def bbody(b, carry):
        its = [item_rows[b, pl.ds(16 * k, 16)] for k in range(4)]
        srow0 = b * NSAMP
        gb = gb0 + b
        vecs = [jnp.zeros((16,), jnp.float32), jnp.zeros((16,), jnp.float32)]
        for l in range(NSAMP):
            srow = srow0 + l
            p = its[0] * samp_rows[srow, pl.ds(0, 16)]
            for k in range(1, 4):
                p = p + its[k] * samp_rows[srow, pl.ds(16 * k, 16)]
            sc = jnp.sum(p)
            vecs[l // 16] = jnp.where(
                iota == (l % 16), jnp.full((16,), sc), vecs[l // 16])
        plsc.store_scatter(out_v, [iota * B_PER_W + gb], vecs[0])
        plsc.store_scatter(out_v, [(16 + iota) * B_PER_W + gb], vecs[1],
                           mask=iota < 4)
        return carry

    lax.fori_loop(0, CHUNK, bbody, 0)


def _item2vec_sc(items, samples_flat, packed):
    mesh = plsc.VectorSubcoreMesh(
        core_axis_name="c", subcore_axis_name="s",
        num_cores=NUM_CORES, num_subcores=NUM_SUBCORES)

    @functools.partial(
        pl.kernel,
        out_type=jax.ShapeDtypeStruct((BATCH * NSAMP,), jnp.float32),
        mesh=mesh,
        scratch_types=[
            pltpu.VMEM((B_PER_W,), jnp.int32),
            pltpu.VMEM((NS_ALL,), jnp.int32),
            pltpu.VMEM((B_PER_W,), jnp.int32),
            pltpu.VMEM((NS_ALL,), jnp.int32),
            [pltpu.VMEM((CHUNK, DIM), jnp.float32)] * 2,
            [pltpu.VMEM((SROWS, DIM), jnp.float32)] * 2,
            pltpu.VMEM((NS_ALL,), jnp.float32),
            [pltpu.SemaphoreType.DMA] * 2,
            pltpu.SemaphoreType.DMA,
        ],
        compiler_params=pltpu.CompilerParams(
            needs_layout_passes=False, use_tc_tiling_on_sc=False),
    )
    def k(items_hbm, samples_hbm, emb_hbm, out_hbm,
          iidx_v, sidx_v, ipix_v, spix_v, irows, srows, obuf, gsems, osem):
        wid = lax.axis_index("s") * NUM_CORES + lax.axis_index("c")
        wbase = wid * B_PER_W
        pltpu.sync_copy(items_hbm.at[pl.ds(wbase, B_PER_W)], iidx_v)
        pltpu.sync_copy(
            samples_hbm.at[pl.ds(wbase * NSAMP, NS_ALL)], sidx_v)

        def packrow(i, _, src, dst):
            off = pl.multiple_of(i * 16, 16)
            v = src[pl.ds(off, 16)]
            dst[pl.ds(off, 16)] = (
                ((v >> LOGW) << LOGW)
                | ((v & (W // 2 - 1)) << 1)
                | ((v >> (LOGW - 1)) & 1))
            return _

        lax.fori_loop(0, B_PER_W // 16,
                      functools.partial(packrow, src=iidx_v, dst=ipix_v), 0)
        lax.fori_loop(0, NS_ALL // 16,
                      functools.partial(packrow, src=sidx_v, dst=spix_v), 0)

        def fire(c, b):
            ibase = pl.multiple_of(c * CHUNK, CHUNK)
            sbase = pl.multiple_of(c * SROWS, SROWS)
            pltpu.make_async_copy(
                emb_hbm.at[ipix_v.at[pl.ds(ibase, CHUNK)]],
                irows[b], gsems[b]).start()
            off = 0
            for n in SSTREAMS:
                pltpu.make_async_copy(
                    emb_hbm.at[spix_v.at[pl.ds(sbase + off, n)]],
                    srows[b].at[pl.ds(off, n)], gsems[b]).start()
                off += n

        def drain(b):
            pltpu.make_async_copy(
                emb_hbm.at[ipix_v.at[pl.ds(0, CHUNK)]],
                irows[b], gsems[b]).wait()
            off = 0
            for n in SSTREAMS:
                pltpu.make_async_copy(
                    emb_hbm.at[spix_v.at[pl.ds(0, n)]],
                    srows[b].at[pl.ds(off, n)], gsems[b]).wait()
                off += n

        def stage(c0, b):
            drain(b)
            _compute_chunk(c0 + b, iidx_v, sidx_v,
                           irows[b], srows[b], obuf)

            @pl.when(c0 + b + 2 < NCHUNK)
            def _():
                fire(c0 + b + 2, b)

        fire(0, 0)
        fire(1, 1)

        def pair_body(kk, carry):
            c0 = kk * 2
            for b in range(2):
                stage(c0, b)
            return carry

        lax.fori_loop(0, NCHUNK // 2, pair_body, 0)
        for l in range(NSAMP):
            pltpu.make_async_copy(
                obuf.at[pl.ds(l * B_PER_W, B_PER_W)],
                out_hbm.at[pl.ds(l * BATCH + wbase, B_PER_W)],
                osem).start()
        for l in range(NSAMP):
            pltpu.make_async_copy(
                obuf.at[pl.ds(0, B_PER_W)],
                out_hbm.at[pl.ds(wbase, B_PER_W)], osem).wait()

    return k(items, samples_flat, packed)


@jax.jit
def kernel(items, samples, embeddings):
    packed = _repack_tc(embeddings.T).reshape(2 * PROWS, DIM)
    scores_flat = _item2vec_sc(items, samples.reshape(-1), packed)
    return scores_flat.reshape(NSAMP, BATCH).T
